# Initial kernel scaffold; baseline (speedup 1.0000x reference)
import functools

import jax
import jax.numpy as jnp
from jax import lax
from jax.experimental import pallas as pl
from jax.experimental.pallas import tpu as pltpu
from jax.experimental.pallas import tpu_sc as plsc

_FIB = (1, 1, 2, 3, 5, 8, 13, 21)
_PRIMES = (2654435761, 2246822519, 3266489917, 2028178513, 1220703125,
           1610612741, 805306457, 402653189, 3674653429, 2860486313,
           1073676287, 2971215073, 1500450271, 3267000013, 2654435789,
           4049292737, 2246822531, 3266489927, 2028178519, 1220703133)

_VOCAB = 1024
_BYTE_DIM = 128
_NUM_TABLES = 8
_BUCKETS = 65536
_EPT = 16
_HIDDEN = 512
_B, _T = 64, 512
_N = _B * _T
_ROWS = _NUM_TABLES * _N
_GROUP = 128
_NGROUPS = _ROWS // _GROUP
_NC, _NS = 2, 16
_NW = _NC * _NS
_GPW = _NGROUPS // _NW
_CHUNK = 16
_NCHUNK = _GPW // _CHUNK
_TILE = 512


def _prime_i32(t, k):
    p = int(_PRIMES[(t * 3 + k) % len(_PRIMES)]) % (1 << 32)
    if p >= 1 << 31:
        p -= 1 << 32
    return jnp.int32(p)


def _hash_idx_body(tok_ref, out_ref):
    tok = tok_ref[...]
    shifted = {}
    for off in sorted(set(_FIB)):
        z = jnp.zeros((_B, off), jnp.int32)
        shifted[off] = jnp.concatenate([z, tok[:, : _T - off]], axis=1)
    for t in range(_NUM_TABLES):
        h = jnp.zeros((_B, _T), jnp.int32)
        for k, off in enumerate(_FIB):
            h = h ^ (shifted[off] * _prime_i32(t, k))
        out_ref[t] = (h & jnp.int32(_BUCKETS - 1)) + jnp.int32(t * _BUCKETS)


_hash_idx = pl.pallas_call(
    _hash_idx_body,
    out_shape=jax.ShapeDtypeStruct((_NUM_TABLES, _B, _T), jnp.int32),
)


_sc_mesh = plsc.VectorSubcoreMesh(
    core_axis_name="c", subcore_axis_name="s",
    num_cores=_NC, num_subcores=_NS)


@functools.partial(
    pl.kernel,
    out_type=jax.ShapeDtypeStruct((_ROWS, _EPT), jnp.float32),
    mesh=_sc_mesh,
    scratch_types=[
        pltpu.VMEM((_CHUNK, _GROUP), jnp.int32),
        pltpu.VMEM((_CHUNK * _GROUP, _EPT), jnp.float32),
        pltpu.SemaphoreType.DMA,
    ],
)
def _sc_gather(tables_hbm, gidx_hbm, out_hbm, idx_v, rows_v, sem):
    wid = lax.axis_index("s") * _NC + lax.axis_index("c")

    def chunk_body(ci, carry):
        g0 = wid * _GPW + ci * _CHUNK
        pltpu.sync_copy(gidx_hbm.at[pl.ds(g0, _CHUNK)], idx_v)
        descs = [
            pltpu.async_copy(
                tables_hbm.at[idx_v.at[j]],
                rows_v.at[pl.ds(j * _GROUP, _GROUP)], sem)
            for j in range(_CHUNK)
        ]
        for d in descs:
            d.wait()
        pltpu.sync_copy(rows_v, out_hbm.at[pl.ds(g0 * _GROUP, _CHUNK * _GROUP)])
        return carry

    lax.fori_loop(0, _NCHUNK, chunk_body, 0)


def _mm_body(tok_ref, bt_ref, xh_ref, w_ref, b_ref, out_ref):
    tok = tok_ref[0]
    iota_v = lax.broadcasted_iota(jnp.int32, (_VOCAB, _TILE), 0)
    oh = (iota_v == tok).astype(jnp.float32)
    be = lax.dot_general(oh, bt_ref[...], (((0,), (0,)), ((), ())),
                         preferred_element_type=jnp.float32)
    xh = xh_ref[...]
    x = jnp.concatenate([be] + [xh[t] for t in range(_NUM_TABLES)], axis=-1)
    out_ref[...] = (
        jnp.dot(x, w_ref[...], preferred_element_type=jnp.float32) + b_ref[...])


_matmul = pl.pallas_call(
    _mm_body,
    grid=(_N // _TILE,),
    in_specs=[
        pl.BlockSpec((1, 1, _TILE), lambda i: (i, 0, 0)),
        pl.BlockSpec((_VOCAB, _BYTE_DIM), lambda i: (0, 0)),
        pl.BlockSpec((_NUM_TABLES, _TILE, _EPT), lambda i: (0, i, 0)),
        pl.BlockSpec((_BYTE_DIM + _NUM_TABLES * _EPT, _HIDDEN), lambda i: (0, 0)),
        pl.BlockSpec((1, _HIDDEN), lambda i: (0, 0)),
    ],
    out_specs=pl.BlockSpec((_TILE, _HIDDEN), lambda i: (i, 0)),
    out_shape=jax.ShapeDtypeStruct((_N, _HIDDEN), jnp.float32),
)


def kernel(tokens, byte_table, hash_tables, W, b):
    gidx = _hash_idx(tokens)
    gidx2 = gidx.reshape(_NGROUPS, _GROUP)
    flat_tables = hash_tables.reshape(_NUM_TABLES * _BUCKETS, _EPT)
    xh = _sc_gather(flat_tables, gidx2)
    xh3 = xh.reshape(_NUM_TABLES, _N, _EPT)
    out = _matmul(tokens.reshape(_N // _TILE, 1, _TILE), byte_table, xh3,
                  W, b.reshape(1, _HIDDEN))
    return out.reshape(_B, _T, _HIDDEN)


# speedup vs baseline: 3.1721x; 3.1721x over previous
import functools

import jax
import jax.numpy as jnp
from jax import lax
from jax.experimental import pallas as pl
from jax.experimental.pallas import tpu as pltpu
from jax.experimental.pallas import tpu_sc as plsc

_FIB = (1, 1, 2, 3, 5, 8, 13, 21)
_PRIMES = (2654435761, 2246822519, 3266489917, 2028178513, 1220703125,
           1610612741, 805306457, 402653189, 3674653429, 2860486313,
           1073676287, 2971215073, 1500450271, 3267000013, 2654435789,
           4049292737, 2246822531, 3266489927, 2028178519, 1220703133)

_VOCAB = 1024
_BYTE_DIM = 128
_NUM_TABLES = 8
_BUCKETS = 65536
_EPT = 16
_HIDDEN = 512
_B, _T = 64, 512
_N = _B * _T
_ROWS = _NUM_TABLES * _N
_GROUP = 128
_NGROUPS = _ROWS // _GROUP
_NC, _NS = 2, 16
_NW = _NC * _NS
_GPW = _NGROUPS // _NW
_CHUNK = 16
_NCHUNK = _GPW // _CHUNK
_TILE = 512


def _prime_i32(t, k):
    p = int(_PRIMES[(t * 3 + k) % len(_PRIMES)]) % (1 << 32)
    if p >= 1 << 31:
        p -= 1 << 32
    return jnp.int32(p)


def _hash_idx_body(tok_ref, out_ref):
    tok = tok_ref[...]
    shifted = {}
    for off in sorted(set(_FIB)):
        z = jnp.zeros((_B, off), jnp.int32)
        shifted[off] = jnp.concatenate([z, tok[:, : _T - off]], axis=1)
    for t in range(_NUM_TABLES):
        h = jnp.zeros((_B, _T), jnp.int32)
        for k, off in enumerate(_FIB):
            h = h ^ (shifted[off] * _prime_i32(t, k))
        out_ref[t] = (h & jnp.int32(_BUCKETS - 1)) + jnp.int32(t * _BUCKETS)


_hash_idx = pl.pallas_call(
    _hash_idx_body,
    out_shape=jax.ShapeDtypeStruct((_NUM_TABLES, _B, _T), jnp.int32),
)


def _sc_gather_body(tables_hbm, gidx_hbm, out_hbm, idx_v, rows_v, sem):
    wid = lax.axis_index("s") * _NC + lax.axis_index("c")

    def chunk_body(ci, carry):
        g0 = wid * _GPW + ci * _CHUNK
        pltpu.sync_copy(gidx_hbm.at[pl.ds(g0, _CHUNK)], idx_v)
        descs = [
            pltpu.async_copy(
                tables_hbm.at[idx_v.at[j]],
                rows_v.at[pl.ds(j * _GROUP, _GROUP)], sem)
            for j in range(_CHUNK)
        ]
        for d in descs:
            d.wait()
        pltpu.sync_copy(rows_v, out_hbm.at[pl.ds(g0 * _GROUP, _CHUNK * _GROUP)])
        return carry

    lax.fori_loop(0, _NCHUNK, chunk_body, 0)


@functools.cache
def _build_sc_gather():
    mesh = plsc.VectorSubcoreMesh(
        core_axis_name="c", subcore_axis_name="s",
        num_cores=_NC, num_subcores=_NS)
    return pl.kernel(
        _sc_gather_body,
        out_type=jax.ShapeDtypeStruct((_ROWS, _EPT), jnp.float32),
        mesh=mesh,
        scratch_types=[
            pltpu.VMEM((_CHUNK, _GROUP), jnp.int32),
            pltpu.VMEM((_CHUNK * _GROUP, _EPT), jnp.float32),
            pltpu.SemaphoreType.DMA,
        ],
        compiler_params=pltpu.CompilerParams(use_tc_tiling_on_sc=False),
    )


def _mm_body(tok_ref, bt_ref, xh_ref, w_ref, b_ref, out_ref):
    tok = tok_ref[0]
    iota_v = lax.broadcasted_iota(jnp.int32, (_VOCAB, _TILE), 0)
    oh = (iota_v == tok).astype(jnp.float32)
    be = lax.dot_general(oh, bt_ref[...], (((0,), (0,)), ((), ())),
                         preferred_element_type=jnp.float32)
    xh = xh_ref[...]
    x = jnp.concatenate([be] + [xh[t] for t in range(_NUM_TABLES)], axis=-1)
    out_ref[...] = (
        jnp.dot(x, w_ref[...], preferred_element_type=jnp.float32) + b_ref[...])


_matmul = pl.pallas_call(
    _mm_body,
    grid=(_N // _TILE,),
    in_specs=[
        pl.BlockSpec((1, 1, _TILE), lambda i: (i, 0, 0)),
        pl.BlockSpec((_VOCAB, _BYTE_DIM), lambda i: (0, 0)),
        pl.BlockSpec((_NUM_TABLES, _TILE, _EPT), lambda i: (0, i, 0)),
        pl.BlockSpec((_BYTE_DIM + _NUM_TABLES * _EPT, _HIDDEN), lambda i: (0, 0)),
        pl.BlockSpec((1, _HIDDEN), lambda i: (0, 0)),
    ],
    out_specs=pl.BlockSpec((_TILE, _HIDDEN), lambda i: (i, 0)),
    out_shape=jax.ShapeDtypeStruct((_N, _HIDDEN), jnp.float32),
)


def kernel(tokens, byte_table, hash_tables, W, b):
    gidx = _hash_idx(tokens)
    gidx2 = gidx.reshape(_NGROUPS, _GROUP)
    flat_tables = hash_tables.reshape(_NUM_TABLES * _BUCKETS, _EPT)
    xh = _build_sc_gather()(flat_tables, gidx2)
    xh3 = xh.reshape(_NUM_TABLES, _N, _EPT)
    out = _matmul(tokens.reshape(_N // _TILE, 1, _TILE), byte_table, xh3,
                  W, b.reshape(1, _HIDDEN))
    return out.reshape(_B, _T, _HIDDEN)


# speedup vs baseline: 4.1787x; 1.3173x over previous
import functools

import jax
import jax.numpy as jnp
from jax import lax
from jax.experimental import pallas as pl
from jax.experimental.pallas import tpu as pltpu
from jax.experimental.pallas import tpu_sc as plsc

_FIB = (1, 1, 2, 3, 5, 8, 13, 21)
_PRIMES = (2654435761, 2246822519, 3266489917, 2028178513, 1220703125,
           1610612741, 805306457, 402653189, 3674653429, 2860486313,
           1073676287, 2971215073, 1500450271, 3267000013, 2654435789,
           4049292737, 2246822531, 3266489927, 2028178519, 1220703133)

_VOCAB = 1024
_BYTE_DIM = 128
_NUM_TABLES = 8
_BUCKETS = 65536
_EPT = 16
_HIDDEN = 512
_B, _T = 64, 512
_N = _B * _T
_ROWS = _NUM_TABLES * _N
_GROUP = 128
_NC, _NS = 2, 16
_NW = _NC * _NS
_TPW = _N // _NW
_TCHUNK = 512
_NCHUNK = _TPW // _TCHUNK
_CROWS = _TCHUNK * _NUM_TABLES
_CGROUPS = _CROWS // _GROUP
_TILE = 512


def _prime_i32(t, k):
    p = int(_PRIMES[(t * 3 + k) % len(_PRIMES)]) % (1 << 32)
    if p >= 1 << 31:
        p -= 1 << 32
    return jnp.int32(p)


def _hash_idx_body(tok_ref, out_ref):
    tok = tok_ref[...]
    shifted = {}
    for off in sorted(set(_FIB)):
        z = jnp.zeros((_B, off), jnp.int32)
        shifted[off] = jnp.concatenate([z, tok[:, : _T - off]], axis=1)
    for t in range(_NUM_TABLES):
        h = jnp.zeros((_B, _T), jnp.int32)
        for k, off in enumerate(_FIB):
            h = h ^ (shifted[off] * _prime_i32(t, k))
        out_ref[t] = (h & jnp.int32(_BUCKETS - 1)) + jnp.int32(t * _BUCKETS)


_hash_idx = pl.pallas_call(
    _hash_idx_body,
    out_shape=jax.ShapeDtypeStruct((_NUM_TABLES, _B, _T), jnp.int32),
)


def _sc_gather_body(tables_hbm, gidx_hbm, out_hbm, idx_raw, rows_v, sem):
    wid = lax.axis_index("s") * _NC + lax.axis_index("c")

    def chunk_body(ci, carry):
        n0 = wid * _TPW + ci * _TCHUNK
        pltpu.sync_copy(gidx_hbm.at[:, pl.ds(n0, _TCHUNK)], idx_raw)
        for half in range(2):
            descs = []
            for t in range(_NUM_TABLES // 2):
                tt = half * (_NUM_TABLES // 2) + t
                for g in range(_TCHUNK // _GROUP):
                    descs.append(pltpu.async_copy(
                        tables_hbm.at[idx_raw.at[tt, pl.ds(g * _GROUP, _GROUP)]],
                        rows_v.at[pl.ds(tt * _TCHUNK + g * _GROUP, _GROUP)],
                        sem))
            for d in descs:
                d.wait()
        for t in range(_NUM_TABLES):
            pltpu.sync_copy(
                rows_v.at[pl.ds(t * _TCHUNK, _TCHUNK)],
                out_hbm.at[pl.ds(n0, _TCHUNK), pl.ds(t * _EPT, _EPT)])
        return carry

    lax.fori_loop(0, _NCHUNK, chunk_body, 0)


@functools.cache
def _build_sc_gather():
    mesh = plsc.VectorSubcoreMesh(
        core_axis_name="c", subcore_axis_name="s",
        num_cores=_NC, num_subcores=_NS)
    return pl.kernel(
        _sc_gather_body,
        out_type=jax.ShapeDtypeStruct((_N, _NUM_TABLES * _EPT), jnp.float32),
        mesh=mesh,
        scratch_types=[
            pltpu.VMEM((_NUM_TABLES, _TCHUNK), jnp.int32),
            pltpu.VMEM((_CROWS, _EPT), jnp.float32),
            pltpu.SemaphoreType.DMA,
        ],
        compiler_params=pltpu.CompilerParams(use_tc_tiling_on_sc=False),
    )


def _mm_body(tok_ref, bt_ref, xh_ref, w_ref, b_ref, out_ref):
    tok = tok_ref[0]
    iota_v = lax.broadcasted_iota(jnp.int32, (_VOCAB, _TILE), 0)
    oh = (iota_v == tok).astype(jnp.float32)
    be = lax.dot_general(oh, bt_ref[...], (((0,), (0,)), ((), ())),
                         preferred_element_type=jnp.float32)
    x = jnp.concatenate([be, xh_ref[...]], axis=-1)
    out_ref[...] = (
        jnp.dot(x, w_ref[...], preferred_element_type=jnp.float32) + b_ref[...])


_matmul = pl.pallas_call(
    _mm_body,
    grid=(_N // _TILE,),
    in_specs=[
        pl.BlockSpec((1, 1, _TILE), lambda i: (i, 0, 0)),
        pl.BlockSpec((_VOCAB, _BYTE_DIM), lambda i: (0, 0)),
        pl.BlockSpec((_TILE, _NUM_TABLES * _EPT), lambda i: (i, 0)),
        pl.BlockSpec((_BYTE_DIM + _NUM_TABLES * _EPT, _HIDDEN), lambda i: (0, 0)),
        pl.BlockSpec((1, _HIDDEN), lambda i: (0, 0)),
    ],
    out_specs=pl.BlockSpec((_TILE, _HIDDEN), lambda i: (i, 0)),
    out_shape=jax.ShapeDtypeStruct((_N, _HIDDEN), jnp.float32),
)


def kernel(tokens, byte_table, hash_tables, W, b):
    gidx = _hash_idx(tokens)
    gidx2 = gidx.reshape(_NUM_TABLES, _N)
    flat_tables = hash_tables.reshape(_NUM_TABLES * _BUCKETS, _EPT)
    xh = _build_sc_gather()(flat_tables, gidx2)
    out = _matmul(tokens.reshape(_N // _TILE, 1, _TILE), byte_table, xh,
                  W, b.reshape(1, _HIDDEN))
    return out.reshape(_B, _T, _HIDDEN)


# speedup vs baseline: 4.2695x; 1.0217x over previous
import functools

import jax
import jax.numpy as jnp
from jax import lax
from jax.experimental import pallas as pl
from jax.experimental.pallas import tpu as pltpu
from jax.experimental.pallas import tpu_sc as plsc

_FIB = (1, 1, 2, 3, 5, 8, 13, 21)
_PRIMES = (2654435761, 2246822519, 3266489917, 2028178513, 1220703125,
           1610612741, 805306457, 402653189, 3674653429, 2860486313,
           1073676287, 2971215073, 1500450271, 3267000013, 2654435789,
           4049292737, 2246822531, 3266489927, 2028178519, 1220703133)

_VOCAB = 1024
_BYTE_DIM = 128
_NUM_TABLES = 8
_BUCKETS = 65536
_EPT = 16
_HIDDEN = 512
_B, _T = 64, 512
_N = _B * _T
_ROWS = _NUM_TABLES * _N
_GROUP = 128
_NC, _NS = 2, 16
_NW = _NC * _NS
_TPW = _N // _NW
_TCHUNK = 512
_NCHUNK = _TPW // _TCHUNK
_CROWS = _TCHUNK * _NUM_TABLES
_CGROUPS = _CROWS // _GROUP
_TILE = 512


def _prime_i32(t, k):
    p = int(_PRIMES[(t * 3 + k) % len(_PRIMES)]) % (1 << 32)
    if p >= 1 << 31:
        p -= 1 << 32
    return jnp.int32(p)


def _hash_idx_body(tok_ref, out_ref):
    tok = tok_ref[...]
    shifted = {}
    for off in sorted(set(_FIB)):
        z = jnp.zeros((_B, off), jnp.int32)
        shifted[off] = jnp.concatenate([z, tok[:, : _T - off]], axis=1)
    for t in range(_NUM_TABLES):
        h = jnp.zeros((_B, _T), jnp.int32)
        for k, off in enumerate(_FIB):
            h = h ^ (shifted[off] * _prime_i32(t, k))
        out_ref[t] = h & jnp.int32(_BUCKETS - 1)


_hash_idx = pl.pallas_call(
    _hash_idx_body,
    out_shape=jax.ShapeDtypeStruct((_NUM_TABLES, _B, _T), jnp.int32),
)


def _sc_gather_body(tables_hbm, gidx_hbm, out_hbm, idx_raw, rows_v, sem):
    wid = lax.axis_index("s") * _NC + lax.axis_index("c")

    def chunk_body(ci, carry):
        n0 = wid * _TPW + ci * _TCHUNK
        pltpu.sync_copy(gidx_hbm.at[:, pl.ds(n0, _TCHUNK)], idx_raw)
        for half in range(2):
            descs = []
            for t in range(_NUM_TABLES // 2):
                tt = half * (_NUM_TABLES // 2) + t
                for g in range(_TCHUNK // _GROUP):
                    descs.append(pltpu.async_copy(
                        tables_hbm.at[tt].at[
                            idx_raw.at[tt, pl.ds(g * _GROUP, _GROUP)]],
                        rows_v.at[pl.ds(tt * _TCHUNK + g * _GROUP, _GROUP)],
                        sem))
            for d in descs:
                d.wait()
        for t in range(_NUM_TABLES):
            pltpu.sync_copy(
                rows_v.at[pl.ds(t * _TCHUNK, _TCHUNK)],
                out_hbm.at[pl.ds(n0, _TCHUNK), pl.ds(t * _EPT, _EPT)])
        return carry

    lax.fori_loop(0, _NCHUNK, chunk_body, 0)


@functools.cache
def _build_sc_gather():
    mesh = plsc.VectorSubcoreMesh(
        core_axis_name="c", subcore_axis_name="s",
        num_cores=_NC, num_subcores=_NS)
    return pl.kernel(
        _sc_gather_body,
        out_type=jax.ShapeDtypeStruct((_N, _NUM_TABLES * _EPT), jnp.float32),
        name="sc_embedding_gather",
        mesh=mesh,
        scratch_types=[
            pltpu.VMEM((_NUM_TABLES, _TCHUNK), jnp.int32),
            pltpu.VMEM((_CROWS, _EPT), jnp.float32),
            pltpu.SemaphoreType.DMA,
        ],
        compiler_params=pltpu.CompilerParams(use_tc_tiling_on_sc=False),
    )


def _mm_body(tok_ref, bt_ref, xh_ref, w_ref, b_ref, out_ref):
    tok = tok_ref[0]
    iota_v = lax.broadcasted_iota(jnp.int32, (_VOCAB, _TILE), 0)
    oh = (iota_v == tok).astype(jnp.float32)
    be = lax.dot_general(oh, bt_ref[...], (((0,), (0,)), ((), ())),
                         preferred_element_type=jnp.float32)
    x = jnp.concatenate([be, xh_ref[...]], axis=-1)
    out_ref[...] = (
        jnp.dot(x, w_ref[...], preferred_element_type=jnp.float32) + b_ref[...])


_matmul = pl.pallas_call(
    _mm_body,
    grid=(_N // _TILE,),
    in_specs=[
        pl.BlockSpec((1, 1, _TILE), lambda i: (i, 0, 0)),
        pl.BlockSpec((_VOCAB, _BYTE_DIM), lambda i: (0, 0)),
        pl.BlockSpec((_TILE, _NUM_TABLES * _EPT), lambda i: (i, 0)),
        pl.BlockSpec((_BYTE_DIM + _NUM_TABLES * _EPT, _HIDDEN), lambda i: (0, 0)),
        pl.BlockSpec((1, _HIDDEN), lambda i: (0, 0)),
    ],
    out_specs=pl.BlockSpec((_TILE, _HIDDEN), lambda i: (i, 0)),
    out_shape=jax.ShapeDtypeStruct((_N, _HIDDEN), jnp.float32),
)


def kernel(tokens, byte_table, hash_tables, W, b):
    gidx = _hash_idx(tokens)
    gidx2 = gidx.reshape(_NUM_TABLES, _N)
    xh = _build_sc_gather()(hash_tables, gidx2)
    out = _matmul(tokens.reshape(_N // _TILE, 1, _TILE), byte_table, xh,
                  W, b.reshape(1, _HIDDEN))
    return out.reshape(_B, _T, _HIDDEN)
